# baseline (speedup 1.0000x reference)
import functools

import jax
import jax.numpy as jnp
from jax import lax
from jax.experimental import pallas as pl
from jax.experimental.pallas import tpu as pltpu
from jax.experimental.pallas import tpu_sc as plsc

_H = 16
_TOPK = 32
_NC = 2
_NS = 16
_NW = _NC * _NS


def _qkv_kernel(x_ref, wq_ref, bq_ref, wk_ref, bk_ref, wv_ref, bv_ref,
                q_ref, k_ref, v_ref, *, nh):
    x = x_ref[...]
    dh = q_ref.shape[2]
    q = jnp.dot(x, wq_ref[...], preferred_element_type=jnp.float32) + bq_ref[...]
    k = jnp.dot(x, wk_ref[...], preferred_element_type=jnp.float32) + bk_ref[...]
    v = jnp.dot(x, wv_ref[...], preferred_element_type=jnp.float32) + bv_ref[...]
    for h in range(nh):
        q_ref[h] = q[:, dh * h:dh * (h + 1)]
        k_ref[h] = k[:, dh * h:dh * (h + 1)]
        v_ref[h] = v[:, dh * h:dh * (h + 1)]


def _scores_kernel(q_ref, k_ref, tok_ref, s_ref, *, scale):
    s = jnp.dot(q_ref[0], k_ref[0].T,
                preferred_element_type=jnp.float32) * scale
    br = s.shape[0]
    nj = s.shape[1] // 128
    for j in range(nj):
        s_ref[:, pl.ds(j, 1)] = s[:, 128 * j:128 * (j + 1)].reshape(
            br // 8, 1, 8, 128)


def _attn_kernel(q_ref, k_ref, v_ref, t_ref, m_ref, o_ref, *, scale):
    s = jnp.dot(q_ref[0], k_ref[0].T,
                preferred_element_type=jnp.float32) * scale
    p = jnp.where(s >= t_ref[0], jnp.exp(s - m_ref[0]), 0.0)
    denom = jnp.sum(p, axis=1, keepdims=True)
    o_ref[0] = jnp.dot(p, v_ref[0],
                       preferred_element_type=jnp.float32) / denom


def _proj_kernel(*refs):
    parts, (w_ref, b_ref, o_ref) = refs[:-3], refs[-3:]
    cols = [p_ref[h] for p_ref in parts for h in range(p_ref.shape[0])]
    a = jnp.concatenate(cols, axis=1)
    o_ref[...] = jnp.dot(a, w_ref[...],
                         preferred_element_type=jnp.float32) + b_ref[...]


def _sort16d(x):
    return lax.rev(lax.sort(x), (0,))


def _rev(x):
    return lax.rev(x, (0,))


def _merge16(a, b):
    rb = _rev(b)
    return _sort16d(jnp.maximum(a, rb)), _sort16d(jnp.minimum(a, rb))


def _merge32top(a0, a1, b0, b1):
    u0 = jnp.maximum(a0, _rev(b1))
    u1 = jnp.maximum(a1, _rev(b0))
    return _sort16d(jnp.maximum(u0, u1)), _sort16d(jnp.minimum(u0, u1))


def _top32_of_16(vs):
    s = [_sort16d(v) for v in vs]
    pairs = [_merge16(s[2 * i], s[2 * i + 1]) for i in range(8)]
    while len(pairs) > 1:
        pairs = [_merge32top(*pairs[2 * i], *pairs[2 * i + 1])
                 for i in range(len(pairs) // 2)]
    return pairs[0]


def _select_row(bufs, b, rr, gid_ref, iota, nj):
    g = []
    for j in range(nj):
        acc = bufs[b, j, rr, pl.ds(0, 16)]
        for k in range(1, 8):
            acc = jnp.maximum(acc, bufs[b, j, rr, pl.ds(16 * k, 16)])
        g.append(acc)
    a0, a1 = _top32_of_16(g)
    big_m = a0[0]
    tg = jnp.full((16,), a1[15], jnp.float32)
    zeros = jnp.zeros((16,), jnp.int32)
    gid_ref[pl.ds(0, 16)] = zeros
    gid_ref[pl.ds(16, 16)] = zeros
    cnt = zeros
    for j in range(nj):
        m = g[j] >= tg
        idx = cnt + plsc.cumsum(jnp.where(m, 1, 0)) - 1
        plsc.store_scatter(gid_ref, [idx], iota + 16 * j,
                           mask=m & (idx < 32))
        cnt = cnt + plsc.all_reduce_population_count(m)
    cand = []
    ib = jnp.full((16,), b, jnp.int32)
    irr = jnp.full((16,), rr, jnp.int32)
    for half in range(2):
        gv = gid_ref[pl.ds(16 * half, 16)]
        jv = lax.shift_right_logical(gv, 4)
        col = gv & 15
        for k in range(8):
            cand.append(plsc.load_gather(bufs, [ib, jv, irr, col + 16 * k]))
    c0, c1 = _top32_of_16(cand)
    return c1[15], big_m


def _make_sc_select(rtot, width):
    nj = width // 128
    rows_pw = rtot // _NW
    trs = rows_pw // 8
    nb = 2
    mesh = plsc.VectorSubcoreMesh(core_axis_name="c", subcore_axis_name="s")

    def body(s_hbm, thr_hbm, m_hbm, bufs, tbuf, mbuf, gid_ref, sem0, sem1):
        wid = lax.axis_index("s") * _NC + lax.axis_index("c")
        base = wid * trs
        iota = lax.iota(jnp.int32, 16)
        sems = (sem0, sem1)
        for b in range(nb):
            pltpu.async_copy(s_hbm.at[base + b], bufs.at[b], sems[b])

        def gbody(gi, carry):
            for b in range(nb):
                chunk = gi * nb + b
                pltpu.make_async_copy(s_hbm.at[base + chunk],
                                      bufs.at[b], sems[b]).wait()

                def rbody(rr, c2):
                    t_val, m_val = _select_row(bufs, b, rr, gid_ref, iota, nj)
                    ridx = jnp.full((16,), chunk * 8 + rr, jnp.int32)
                    lane0 = iota == 0
                    plsc.store_scatter(tbuf, [ridx],
                                       jnp.full((16,), t_val, jnp.float32),
                                       mask=lane0)
                    plsc.store_scatter(mbuf, [ridx],
                                       jnp.full((16,), m_val, jnp.float32),
                                       mask=lane0)
                    return c2

                lax.fori_loop(0, 8, rbody, 0)
                nxt = chunk + nb

                @pl.when(nxt < trs)
                def _():
                    pltpu.async_copy(s_hbm.at[base + nxt],
                                     bufs.at[b], sems[b])
            return carry

        lax.fori_loop(0, trs // nb, gbody, 0)
        pltpu.sync_copy(tbuf, thr_hbm.at[pl.ds(wid * rows_pw, rows_pw)])
        pltpu.sync_copy(mbuf, m_hbm.at[pl.ds(wid * rows_pw, rows_pw)])

    return pl.kernel(
        body,
        out_type=[jax.ShapeDtypeStruct((rtot,), jnp.float32),
                  jax.ShapeDtypeStruct((rtot,), jnp.float32)],
        mesh=mesh,
        compiler_params=pltpu.CompilerParams(needs_layout_passes=False),
        scratch_types=[
            pltpu.VMEM((nb, nj, 8, 128), jnp.float32),
            pltpu.VMEM((rows_pw,), jnp.float32),
            pltpu.VMEM((rows_pw,), jnp.float32),
            pltpu.VMEM((32,), jnp.int32),
            pltpu.SemaphoreType.DMA,
            pltpu.SemaphoreType.DMA,
        ],
    )


def kernel(x, Wq, bq, Wk, bk, Wv, bv, Wo, bo):
    Bv, N, Din = x.shape
    Dout = Wq.shape[1]
    dh = Dout // _H
    scale = 1.0 / (dh ** 0.5)
    x2 = x.reshape(Bv * N, Din)

    br_qkv = 256
    q3, k3, v3 = pl.pallas_call(
        functools.partial(_qkv_kernel, nh=_H),
        grid=(N // br_qkv,),
        in_specs=[
            pl.BlockSpec((br_qkv, Din), lambda r: (r, 0)),
            pl.BlockSpec((Din, Dout), lambda r: (0, 0)),
            pl.BlockSpec((1, Dout), lambda r: (0, 0)),
            pl.BlockSpec((Din, Dout), lambda r: (0, 0)),
            pl.BlockSpec((1, Dout), lambda r: (0, 0)),
            pl.BlockSpec((Din, Dout), lambda r: (0, 0)),
            pl.BlockSpec((1, Dout), lambda r: (0, 0)),
        ],
        out_specs=[
            pl.BlockSpec((_H, br_qkv, dh), lambda r: (0, r, 0)),
            pl.BlockSpec((_H, br_qkv, dh), lambda r: (0, r, 0)),
            pl.BlockSpec((_H, br_qkv, dh), lambda r: (0, r, 0)),
        ],
        out_shape=[jax.ShapeDtypeStruct((_H, N, dh), jnp.float32)] * 3,
    )(x2, Wq, bq.reshape(1, Dout), Wk, bk.reshape(1, Dout),
      Wv, bv.reshape(1, Dout))

    br = 128
    nr = N // br
    ng = 16
    hg = _H // ng
    parts = []
    for gi in range(ng):
        q3g = q3[gi * hg:(gi + 1) * hg]
        k3g = k3[gi * hg:(gi + 1) * hg]
        v3g = v3[gi * hg:(gi + 1) * hg]
        tok = parts[gi - 2] if gi >= 2 else q3g
        scores = pl.pallas_call(
            functools.partial(_scores_kernel, scale=scale),
            grid=(hg, nr),
            in_specs=[
                pl.BlockSpec((1, br, dh), lambda h, r: (h, r, 0)),
                pl.BlockSpec((1, N, dh), lambda h, r: (h, 0, 0)),
                pl.BlockSpec((1, 8, dh), lambda h, r: (0, 0, 0)),
            ],
            out_specs=pl.BlockSpec((br // 8, N // 128, 8, 128),
                                   lambda h, r: (h * nr + r, 0, 0, 0)),
            out_shape=jax.ShapeDtypeStruct((hg * N // 8, N // 128, 8, 128),
                                           jnp.float32),
        )(q3g, k3g, tok)

        thr, rowmax = _make_sc_select(hg * N, N)(scores)
        thr3 = thr.reshape(hg, N, 1)
        m3 = rowmax.reshape(hg, N, 1)

        parts.append(pl.pallas_call(
            functools.partial(_attn_kernel, scale=scale),
            grid=(hg, nr),
            in_specs=[
                pl.BlockSpec((1, br, dh), lambda h, r: (h, r, 0)),
                pl.BlockSpec((1, N, dh), lambda h, r: (h, 0, 0)),
                pl.BlockSpec((1, N, dh), lambda h, r: (h, 0, 0)),
                pl.BlockSpec((1, br, 1), lambda h, r: (h, r, 0)),
                pl.BlockSpec((1, br, 1), lambda h, r: (h, r, 0)),
            ],
            out_specs=pl.BlockSpec((1, br, dh), lambda h, r: (h, r, 0)),
            out_shape=jax.ShapeDtypeStruct((hg, N, dh), jnp.float32),
        )(q3g, k3g, v3g, thr3, m3))

    br_o = 256
    out = pl.pallas_call(
        _proj_kernel,
        grid=(N // br_o,),
        in_specs=(
            [pl.BlockSpec((hg, br_o, dh), lambda r: (0, r, 0))] * ng + [
                pl.BlockSpec((Dout, Dout), lambda r: (0, 0)),
                pl.BlockSpec((1, Dout), lambda r: (0, 0)),
            ]),
        out_specs=pl.BlockSpec((br_o, Dout), lambda r: (r, 0)),
        out_shape=jax.ShapeDtypeStruct((N, Dout), jnp.float32),
    )(*parts, Wo, bo.reshape(1, Dout))

    return out.reshape(Bv, N, Dout)


# speedup vs baseline: 1.2815x; 1.2815x over previous
import functools

import jax
import jax.numpy as jnp
from jax import lax
from jax.experimental import pallas as pl
from jax.experimental.pallas import tpu as pltpu
from jax.experimental.pallas import tpu_sc as plsc

_H = 16
_TOPK = 32
_NC = 2
_NS = 16
_NW = _NC * _NS


def _qkv_kernel(x_ref, wq_ref, bq_ref, wk_ref, bk_ref, wv_ref, bv_ref,
                q_ref, k_ref, v_ref, *, nh):
    x = x_ref[...]
    dh = q_ref.shape[2]
    q = jnp.dot(x, wq_ref[...], preferred_element_type=jnp.float32) + bq_ref[...]
    k = jnp.dot(x, wk_ref[...], preferred_element_type=jnp.float32) + bk_ref[...]
    v = jnp.dot(x, wv_ref[...], preferred_element_type=jnp.float32) + bv_ref[...]
    for h in range(nh):
        q_ref[h] = q[:, dh * h:dh * (h + 1)]
        k_ref[h] = k[:, dh * h:dh * (h + 1)]
        v_ref[h] = v[:, dh * h:dh * (h + 1)]


def _scores_kernel(q_ref, k_ref, tok_ref, s_ref, *, scale):
    s = jnp.dot(q_ref[0], k_ref[0].T,
                preferred_element_type=jnp.float32) * scale
    br = s.shape[0]
    nj = s.shape[1] // 128
    for j in range(nj):
        s_ref[:, pl.ds(j, 1)] = s[:, 128 * j:128 * (j + 1)].reshape(
            br // 8, 1, 8, 128)


def _attn_kernel(q_ref, k_ref, v_ref, t_ref, m_ref, o_ref, *, scale):
    s = jnp.dot(q_ref[0], k_ref[0].T,
                preferred_element_type=jnp.float32) * scale
    p = jnp.where(s >= t_ref[0], jnp.exp(s - m_ref[0]), 0.0)
    denom = jnp.sum(p, axis=1, keepdims=True)
    o_ref[0] = jnp.dot(p, v_ref[0],
                       preferred_element_type=jnp.float32) / denom


def _proj_kernel(*refs):
    parts, (w_ref, b_ref, o_ref) = refs[:-3], refs[-3:]
    cols = [p_ref[h] for p_ref in parts for h in range(p_ref.shape[0])]
    a = jnp.concatenate(cols, axis=1)
    o_ref[...] = jnp.dot(a, w_ref[...],
                         preferred_element_type=jnp.float32) + b_ref[...]


def _sort16d(x):
    return lax.rev(lax.sort(x), (0,))


def _rev(x):
    return lax.rev(x, (0,))


def _merge16(a, b):
    rb = _rev(b)
    return _sort16d(jnp.maximum(a, rb)), _sort16d(jnp.minimum(a, rb))


def _merge32top(a0, a1, b0, b1):
    u0 = jnp.maximum(a0, _rev(b1))
    u1 = jnp.maximum(a1, _rev(b0))
    return _sort16d(jnp.maximum(u0, u1)), _sort16d(jnp.minimum(u0, u1))


def _top32_of_16(vs):
    s = [_sort16d(v) for v in vs]
    pairs = [_merge16(s[2 * i], s[2 * i + 1]) for i in range(8)]
    while len(pairs) > 1:
        pairs = [_merge32top(*pairs[2 * i], *pairs[2 * i + 1])
                 for i in range(len(pairs) // 2)]
    return pairs[0]


def _select_row(bufs, b, rr, gid_ref, iota, nj):
    g = []
    for j in range(nj):
        acc = bufs[b, j, rr, pl.ds(0, 16)]
        for k in range(1, 8):
            acc = jnp.maximum(acc, bufs[b, j, rr, pl.ds(16 * k, 16)])
        g.append(acc)
    a0, a1 = _top32_of_16(g)
    big_m = a0[0]
    tg = jnp.full((16,), a1[15], jnp.float32)
    zeros = jnp.zeros((16,), jnp.int32)
    gid_ref[pl.ds(0, 16)] = zeros
    gid_ref[pl.ds(16, 16)] = zeros
    cnt = zeros
    for j in range(nj):
        m = g[j] >= tg
        idx = cnt + plsc.cumsum(jnp.where(m, 1, 0)) - 1
        plsc.store_scatter(gid_ref, [idx], iota + 16 * j,
                           mask=m & (idx < 32))
        cnt = cnt + plsc.all_reduce_population_count(m)
    cand = []
    ib = jnp.full((16,), b, jnp.int32)
    irr = jnp.full((16,), rr, jnp.int32)
    for half in range(2):
        gv = gid_ref[pl.ds(16 * half, 16)]
        jv = lax.shift_right_logical(gv, 4)
        col = gv & 15
        for k in range(8):
            cand.append(plsc.load_gather(bufs, [ib, jv, irr, col + 16 * k]))
    c0, c1 = _top32_of_16(cand)
    return c1[15], big_m


def _make_sc_select(rtot, width):
    nj = width // 128
    rows_pw = rtot // _NW
    trs = rows_pw // 8
    nb = 2
    mesh = plsc.VectorSubcoreMesh(core_axis_name="c", subcore_axis_name="s")

    def body(s_hbm, thr_hbm, m_hbm, bufs, tbuf, mbuf, gid_ref, sem0, sem1):
        wid = lax.axis_index("s") * _NC + lax.axis_index("c")
        base = wid * trs
        iota = lax.iota(jnp.int32, 16)
        sems = (sem0, sem1)
        for b in range(nb):
            pltpu.async_copy(s_hbm.at[base + b], bufs.at[b], sems[b])

        def gbody(gi, carry):
            for b in range(nb):
                chunk = gi * nb + b
                pltpu.make_async_copy(s_hbm.at[base + chunk],
                                      bufs.at[b], sems[b]).wait()

                def rbody(rr, c2):
                    t_val, m_val = _select_row(bufs, b, rr, gid_ref, iota, nj)
                    ridx = jnp.full((16,), chunk * 8 + rr, jnp.int32)
                    lane0 = iota == 0
                    plsc.store_scatter(tbuf, [ridx],
                                       jnp.full((16,), t_val, jnp.float32),
                                       mask=lane0)
                    plsc.store_scatter(mbuf, [ridx],
                                       jnp.full((16,), m_val, jnp.float32),
                                       mask=lane0)
                    return c2

                lax.fori_loop(0, 8, rbody, 0)
                nxt = chunk + nb

                @pl.when(nxt < trs)
                def _():
                    pltpu.async_copy(s_hbm.at[base + nxt],
                                     bufs.at[b], sems[b])
            return carry

        lax.fori_loop(0, trs // nb, gbody, 0)
        pltpu.sync_copy(tbuf, thr_hbm.at[pl.ds(wid * rows_pw, rows_pw)])
        pltpu.sync_copy(mbuf, m_hbm.at[pl.ds(wid * rows_pw, rows_pw)])

    return pl.kernel(
        body,
        out_type=[jax.ShapeDtypeStruct((rtot,), jnp.float32),
                  jax.ShapeDtypeStruct((rtot,), jnp.float32)],
        mesh=mesh,
        compiler_params=pltpu.CompilerParams(needs_layout_passes=False),
        scratch_types=[
            pltpu.VMEM((nb, nj, 8, 128), jnp.float32),
            pltpu.VMEM((rows_pw,), jnp.float32),
            pltpu.VMEM((rows_pw,), jnp.float32),
            pltpu.VMEM((32,), jnp.int32),
            pltpu.SemaphoreType.DMA,
            pltpu.SemaphoreType.DMA,
        ],
    )


def kernel(x, Wq, bq, Wk, bk, Wv, bv, Wo, bo):
    Bv, N, Din = x.shape
    Dout = Wq.shape[1]
    dh = Dout // _H
    scale = 1.0 / (dh ** 0.5)
    x2 = x.reshape(Bv * N, Din)

    br_qkv = 256
    q3, k3, v3 = pl.pallas_call(
        functools.partial(_qkv_kernel, nh=_H),
        grid=(N // br_qkv,),
        in_specs=[
            pl.BlockSpec((br_qkv, Din), lambda r: (r, 0)),
            pl.BlockSpec((Din, Dout), lambda r: (0, 0)),
            pl.BlockSpec((1, Dout), lambda r: (0, 0)),
            pl.BlockSpec((Din, Dout), lambda r: (0, 0)),
            pl.BlockSpec((1, Dout), lambda r: (0, 0)),
            pl.BlockSpec((Din, Dout), lambda r: (0, 0)),
            pl.BlockSpec((1, Dout), lambda r: (0, 0)),
        ],
        out_specs=[
            pl.BlockSpec((_H, br_qkv, dh), lambda r: (0, r, 0)),
            pl.BlockSpec((_H, br_qkv, dh), lambda r: (0, r, 0)),
            pl.BlockSpec((_H, br_qkv, dh), lambda r: (0, r, 0)),
        ],
        out_shape=[jax.ShapeDtypeStruct((_H, N, dh), jnp.float32)] * 3,
    )(x2, Wq, bq.reshape(1, Dout), Wk, bk.reshape(1, Dout),
      Wv, bv.reshape(1, Dout))

    br = 256
    nr = N // br
    ng = 8
    hg = _H // ng
    parts = []
    for gi in range(ng):
        q3g = q3[gi * hg:(gi + 1) * hg]
        k3g = k3[gi * hg:(gi + 1) * hg]
        v3g = v3[gi * hg:(gi + 1) * hg]
        tok = parts[gi - 2] if gi >= 2 else q3g
        scores = pl.pallas_call(
            functools.partial(_scores_kernel, scale=scale),
            grid=(hg, nr),
            in_specs=[
                pl.BlockSpec((1, br, dh), lambda h, r: (h, r, 0)),
                pl.BlockSpec((1, N, dh), lambda h, r: (h, 0, 0)),
                pl.BlockSpec((1, 8, dh), lambda h, r: (0, 0, 0)),
            ],
            out_specs=pl.BlockSpec((br // 8, N // 128, 8, 128),
                                   lambda h, r: (h * nr + r, 0, 0, 0)),
            out_shape=jax.ShapeDtypeStruct((hg * N // 8, N // 128, 8, 128),
                                           jnp.float32),
        )(q3g, k3g, tok)

        thr, rowmax = _make_sc_select(hg * N, N)(scores)
        thr3 = thr.reshape(hg, N, 1)
        m3 = rowmax.reshape(hg, N, 1)

        parts.append(pl.pallas_call(
            functools.partial(_attn_kernel, scale=scale),
            grid=(hg, nr),
            in_specs=[
                pl.BlockSpec((1, br, dh), lambda h, r: (h, r, 0)),
                pl.BlockSpec((1, N, dh), lambda h, r: (h, 0, 0)),
                pl.BlockSpec((1, N, dh), lambda h, r: (h, 0, 0)),
                pl.BlockSpec((1, br, 1), lambda h, r: (h, r, 0)),
                pl.BlockSpec((1, br, 1), lambda h, r: (h, r, 0)),
            ],
            out_specs=pl.BlockSpec((1, br, dh), lambda h, r: (h, r, 0)),
            out_shape=jax.ShapeDtypeStruct((hg, N, dh), jnp.float32),
        )(q3g, k3g, v3g, thr3, m3))

    br_o = 256
    out = pl.pallas_call(
        _proj_kernel,
        grid=(N // br_o,),
        in_specs=(
            [pl.BlockSpec((hg, br_o, dh), lambda r: (0, r, 0))] * ng + [
                pl.BlockSpec((Dout, Dout), lambda r: (0, 0)),
                pl.BlockSpec((1, Dout), lambda r: (0, 0)),
            ]),
        out_specs=pl.BlockSpec((br_o, Dout), lambda r: (r, 0)),
        out_shape=jax.ShapeDtypeStruct((N, Dout), jnp.float32),
    )(*parts, Wo, bo.reshape(1, Dout))

    return out.reshape(Bv, N, Dout)


# speedup vs baseline: 1.3722x; 1.0708x over previous
import functools

import jax
import jax.numpy as jnp
from jax import lax
from jax.experimental import pallas as pl
from jax.experimental.pallas import tpu as pltpu
from jax.experimental.pallas import tpu_sc as plsc

_H = 16
_TOPK = 32
_NC = 2
_NS = 16
_NW = _NC * _NS


def _qkv_kernel(x_ref, wq_ref, bq_ref, wk_ref, bk_ref, wv_ref, bv_ref,
                q_ref, k_ref, v_ref, *, nh):
    x = x_ref[...]
    dh = q_ref.shape[2]
    q = jnp.dot(x, wq_ref[...], preferred_element_type=jnp.float32) + bq_ref[...]
    k = jnp.dot(x, wk_ref[...], preferred_element_type=jnp.float32) + bk_ref[...]
    v = jnp.dot(x, wv_ref[...], preferred_element_type=jnp.float32) + bv_ref[...]
    for h in range(nh):
        q_ref[h] = q[:, dh * h:dh * (h + 1)]
        k_ref[h] = k[:, dh * h:dh * (h + 1)]
        v_ref[h] = v[:, dh * h:dh * (h + 1)]


def _scores_kernel(q_ref, k_ref, tok_ref, s_ref, *, scale):
    s = jnp.dot(q_ref[0], k_ref[0].T,
                preferred_element_type=jnp.float32) * scale
    br = s.shape[0]
    nj = s.shape[1] // 128
    for j in range(nj):
        s_ref[:, pl.ds(j, 1)] = s[:, 128 * j:128 * (j + 1)].reshape(
            br // 8, 1, 8, 128)


def _attn_kernel(q_ref, k_ref, v_ref, t_ref, m_ref, o_ref, *, scale):
    s = jnp.dot(q_ref[0], k_ref[0].T,
                preferred_element_type=jnp.float32) * scale
    p = jnp.where(s >= t_ref[0], jnp.exp(s - m_ref[0]), 0.0)
    denom = jnp.sum(p, axis=1, keepdims=True)
    o_ref[0] = jnp.dot(p, v_ref[0],
                       preferred_element_type=jnp.float32) / denom


def _proj_kernel(*refs):
    parts, (w_ref, b_ref, o_ref) = refs[:-3], refs[-3:]
    cols = [p_ref[h] for p_ref in parts for h in range(p_ref.shape[0])]
    a = jnp.concatenate(cols, axis=1)
    o_ref[...] = jnp.dot(a, w_ref[...],
                         preferred_element_type=jnp.float32) + b_ref[...]


def _sort16d(x):
    return lax.rev(lax.sort(x), (0,))


def _rev(x):
    return lax.rev(x, (0,))


def _merge16(a, b):
    rb = _rev(b)
    return _sort16d(jnp.maximum(a, rb)), _sort16d(jnp.minimum(a, rb))


def _merge32top(a0, a1, b0, b1):
    u0 = jnp.maximum(a0, _rev(b1))
    u1 = jnp.maximum(a1, _rev(b0))
    return _sort16d(jnp.maximum(u0, u1)), _sort16d(jnp.minimum(u0, u1))


def _top32_of_16(vs):
    s = [_sort16d(v) for v in vs]
    pairs = [_merge16(s[2 * i], s[2 * i + 1]) for i in range(8)]
    while len(pairs) > 1:
        pairs = [_merge32top(*pairs[2 * i], *pairs[2 * i + 1])
                 for i in range(len(pairs) // 2)]
    return pairs[0]


def _select_row(bufs, b, rr, gid_ref, iota, nj):
    g = []
    for j in range(nj):
        acc = bufs[b, j, rr, pl.ds(0, 16)]
        for k in range(1, 8):
            acc = jnp.maximum(acc, bufs[b, j, rr, pl.ds(16 * k, 16)])
        g.append(acc)
    a0, a1 = _top32_of_16(g)
    big_m = a0[0]
    tg = jnp.full((16,), a1[15], jnp.float32)
    zeros = jnp.zeros((16,), jnp.int32)
    gid_ref[pl.ds(0, 16)] = zeros
    gid_ref[pl.ds(16, 16)] = zeros
    cnt = zeros
    for j in range(nj):
        m = g[j] >= tg
        idx = cnt + plsc.cumsum(jnp.where(m, 1, 0)) - 1
        plsc.store_scatter(gid_ref, [idx], iota + 16 * j,
                           mask=m & (idx < 32))
        cnt = cnt + plsc.all_reduce_population_count(m)
    cand = []
    ib = jnp.full((16,), b, jnp.int32)
    irr = jnp.full((16,), rr, jnp.int32)
    for half in range(2):
        gv = gid_ref[pl.ds(16 * half, 16)]
        jv = lax.shift_right_logical(gv, 4)
        col = gv & 15
        for k in range(8):
            cand.append(plsc.load_gather(bufs, [ib, jv, irr, col + 16 * k]))
    c0, c1 = _top32_of_16(cand)
    return c1[15], big_m


def _make_sc_select(rtot, width):
    nj = width // 128
    rows_pw = rtot // _NW
    trs = rows_pw // 8
    nb = 2
    mesh = plsc.VectorSubcoreMesh(core_axis_name="c", subcore_axis_name="s")

    def body(s_hbm, thr_hbm, m_hbm, bufs, tbuf, mbuf, gid_ref, sem0, sem1):
        wid = lax.axis_index("s") * _NC + lax.axis_index("c")
        base = wid * trs
        iota = lax.iota(jnp.int32, 16)
        sems = (sem0, sem1)
        for b in range(nb):
            pltpu.async_copy(s_hbm.at[base + b], bufs.at[b], sems[b])

        def gbody(gi, carry):
            for b in range(nb):
                chunk = gi * nb + b
                pltpu.make_async_copy(s_hbm.at[base + chunk],
                                      bufs.at[b], sems[b]).wait()

                def rbody(rr, c2):
                    t_val, m_val = _select_row(bufs, b, rr, gid_ref, iota, nj)
                    ridx = jnp.full((16,), chunk * 8 + rr, jnp.int32)
                    lane0 = iota == 0
                    plsc.store_scatter(tbuf, [ridx],
                                       jnp.full((16,), t_val, jnp.float32),
                                       mask=lane0)
                    plsc.store_scatter(mbuf, [ridx],
                                       jnp.full((16,), m_val, jnp.float32),
                                       mask=lane0)
                    return c2

                lax.fori_loop(0, 8, rbody, 0)
                nxt = chunk + nb

                @pl.when(nxt < trs)
                def _():
                    pltpu.async_copy(s_hbm.at[base + nxt],
                                     bufs.at[b], sems[b])
            return carry

        lax.fori_loop(0, trs // nb, gbody, 0)
        pltpu.sync_copy(tbuf, thr_hbm.at[pl.ds(wid * rows_pw, rows_pw)])
        pltpu.sync_copy(mbuf, m_hbm.at[pl.ds(wid * rows_pw, rows_pw)])

    return pl.kernel(
        body,
        out_type=[jax.ShapeDtypeStruct((rtot,), jnp.float32),
                  jax.ShapeDtypeStruct((rtot,), jnp.float32)],
        mesh=mesh,
        compiler_params=pltpu.CompilerParams(needs_layout_passes=False),
        scratch_types=[
            pltpu.VMEM((nb, nj, 8, 128), jnp.float32),
            pltpu.VMEM((rows_pw,), jnp.float32),
            pltpu.VMEM((rows_pw,), jnp.float32),
            pltpu.VMEM((32,), jnp.int32),
            pltpu.SemaphoreType.DMA,
            pltpu.SemaphoreType.DMA,
        ],
    )


def kernel(x, Wq, bq, Wk, bk, Wv, bv, Wo, bo):
    Bv, N, Din = x.shape
    Dout = Wq.shape[1]
    dh = Dout // _H
    scale = 1.0 / (dh ** 0.5)
    x2 = x.reshape(Bv * N, Din)

    br_qkv = 256
    q3, k3, v3 = pl.pallas_call(
        functools.partial(_qkv_kernel, nh=_H),
        grid=(N // br_qkv,),
        in_specs=[
            pl.BlockSpec((br_qkv, Din), lambda r: (r, 0)),
            pl.BlockSpec((Din, Dout), lambda r: (0, 0)),
            pl.BlockSpec((1, Dout), lambda r: (0, 0)),
            pl.BlockSpec((Din, Dout), lambda r: (0, 0)),
            pl.BlockSpec((1, Dout), lambda r: (0, 0)),
            pl.BlockSpec((Din, Dout), lambda r: (0, 0)),
            pl.BlockSpec((1, Dout), lambda r: (0, 0)),
        ],
        out_specs=[
            pl.BlockSpec((_H, br_qkv, dh), lambda r: (0, r, 0)),
            pl.BlockSpec((_H, br_qkv, dh), lambda r: (0, r, 0)),
            pl.BlockSpec((_H, br_qkv, dh), lambda r: (0, r, 0)),
        ],
        out_shape=[jax.ShapeDtypeStruct((_H, N, dh), jnp.float32)] * 3,
    )(x2, Wq, bq.reshape(1, Dout), Wk, bk.reshape(1, Dout),
      Wv, bv.reshape(1, Dout))

    br = 512
    nr = N // br
    ng = 8
    hg = _H // ng
    parts = []
    for gi in range(ng):
        q3g = q3[gi * hg:(gi + 1) * hg]
        k3g = k3[gi * hg:(gi + 1) * hg]
        v3g = v3[gi * hg:(gi + 1) * hg]
        tok = parts[gi - 2] if gi >= 2 else q3g
        scores = pl.pallas_call(
            functools.partial(_scores_kernel, scale=scale),
            grid=(hg, nr),
            in_specs=[
                pl.BlockSpec((1, br, dh), lambda h, r: (h, r, 0)),
                pl.BlockSpec((1, N, dh), lambda h, r: (h, 0, 0)),
                pl.BlockSpec((1, 8, dh), lambda h, r: (0, 0, 0)),
            ],
            out_specs=pl.BlockSpec((br // 8, N // 128, 8, 128),
                                   lambda h, r: (h * nr + r, 0, 0, 0)),
            out_shape=jax.ShapeDtypeStruct((hg * N // 8, N // 128, 8, 128),
                                           jnp.float32),
        )(q3g, k3g, tok)

        thr, rowmax = _make_sc_select(hg * N, N)(scores)
        thr3 = thr.reshape(hg, N, 1)
        m3 = rowmax.reshape(hg, N, 1)

        parts.append(pl.pallas_call(
            functools.partial(_attn_kernel, scale=scale),
            grid=(hg, nr),
            in_specs=[
                pl.BlockSpec((1, br, dh), lambda h, r: (h, r, 0)),
                pl.BlockSpec((1, N, dh), lambda h, r: (h, 0, 0)),
                pl.BlockSpec((1, N, dh), lambda h, r: (h, 0, 0)),
                pl.BlockSpec((1, br, 1), lambda h, r: (h, r, 0)),
                pl.BlockSpec((1, br, 1), lambda h, r: (h, r, 0)),
            ],
            out_specs=pl.BlockSpec((1, br, dh), lambda h, r: (h, r, 0)),
            out_shape=jax.ShapeDtypeStruct((hg, N, dh), jnp.float32),
        )(q3g, k3g, v3g, thr3, m3))

    br_o = 256
    out = pl.pallas_call(
        _proj_kernel,
        grid=(N // br_o,),
        in_specs=(
            [pl.BlockSpec((hg, br_o, dh), lambda r: (0, r, 0))] * ng + [
                pl.BlockSpec((Dout, Dout), lambda r: (0, 0)),
                pl.BlockSpec((1, Dout), lambda r: (0, 0)),
            ]),
        out_specs=pl.BlockSpec((br_o, Dout), lambda r: (r, 0)),
        out_shape=jax.ShapeDtypeStruct((N, Dout), jnp.float32),
    )(*parts, Wo, bo.reshape(1, Dout))

    return out.reshape(Bv, N, Dout)
